# baseline (speedup 1.0000x reference)
import functools

import jax
import jax.numpy as jnp
from jax import lax
from jax.experimental import pallas as pl
from jax.experimental.pallas import tpu as pltpu
from jax.experimental.pallas import tpu_sc as plsc

N = 10000
D = 128
HD = 64
NP = 10240
E = 320000
NTILES = 16
RB = 128
CHUNKS = 160
EPT = CHUNKS * RB
EP = EPT * NTILES
CHK = NP // NTILES
KC = 2
EC = KC * RB
STEPS = CHUNKS // KC
SCH = 2
BR = 1000
NBLK = N // BR

_f32 = jnp.float32


def _dense_body(with_act, xsl_ref, xsh_ref, xdl_ref, xdh_ref, bs_ref, bd_ref,
                Ws_ref, as_ref, Wd_ref, ad_ref, hsl_ref, hsh_ref, als_ref,
                ald_ref, C_ref, mxs_ref, mxd_ref):
    i = pl.program_id(0)
    xs = jnp.concatenate([xsl_ref[...], xsh_ref[...]], axis=1)
    xd = jnp.concatenate([xdl_ref[...], xdh_ref[...]], axis=1)
    if with_act:
        xs = jnp.maximum(xs + bs_ref[...], 0.0)
        xd = jnp.maximum(xd + bd_ref[...], 0.0)
    hs = jnp.dot(xs, Ws_ref[...], preferred_element_type=_f32)
    hsl_ref[...] = hs[:, :HD]
    hsh_ref[...] = hs[:, HD:]
    hd = jnp.dot(xd, Wd_ref[...], preferred_element_type=_f32)
    als = jnp.sum(hs * as_ref[...], axis=1, keepdims=True)
    ald = jnp.sum(hd * ad_ref[...], axis=1, keepdims=True)
    als_ref[...] = als
    ald_ref[...] = ald

    @pl.when(i == 0)
    def _():
        mxs_ref[...] = jnp.full((8, 128), -3e38, _f32)
        mxd_ref[...] = jnp.full((8, 128), -3e38, _f32)

    mxs_ref[...] = jnp.maximum(mxs_ref[...], jnp.max(als))
    mxd_ref[...] = jnp.maximum(mxd_ref[...], jnp.max(ald))

    @pl.when(i == NBLK - 1)
    def _():
        C_ref[...] = jnp.zeros((8, 128), _f32) + (jnp.max(mxs_ref[...]) +
                                                  jnp.max(mxd_ref[...]))


def _dense_pre(xs_lo, xs_hi, xd_lo, xd_hi, b_src, b_dst, Ws, a_s, Wd, a_d,
               with_act):
    full = lambda s: pl.BlockSpec(s, lambda i: (0, 0))
    half = pl.BlockSpec((BR, HD), lambda i: (i, 0))
    hsl, hsh, als, ald, C = pl.pallas_call(
        functools.partial(_dense_body, with_act),
        grid=(NBLK,),
        in_specs=[half, half, half, half,
                  full((1, D)), full((1, D)), full((D, D)), full((1, D)),
                  full((D, D)), full((1, D))],
        out_specs=[half, half,
                   pl.BlockSpec((BR, 1), lambda i: (i, 0)),
                   pl.BlockSpec((BR, 1), lambda i: (i, 0)),
                   full((8, 128))],
        out_shape=[
            jax.ShapeDtypeStruct((N, HD), _f32),
            jax.ShapeDtypeStruct((N, HD), _f32),
            jax.ShapeDtypeStruct((N, 1), _f32),
            jax.ShapeDtypeStruct((N, 1), _f32),
            jax.ShapeDtypeStruct((8, 128), _f32),
        ],
        scratch_shapes=[pltpu.VMEM((8, 128), _f32), pltpu.VMEM((8, 128), _f32)],
    )(xs_lo, xs_hi, xd_lo, xd_hi, b_src.reshape(1, D), b_dst.reshape(1, D),
      Ws, a_s.reshape(1, D), Wd, a_d.reshape(1, D))
    return hsl, hsh, als.reshape(N), ald.reshape(N), C[0, :16]


def _final_body(mul_ref, muh_ref, bu_ref, Wou_ref, bou_ref, mil_ref, mih_ref,
                bi_ref, Woi_ref, boi_ref, ou_ref, oi_ref):
    xu = jnp.concatenate([mul_ref[...], muh_ref[...]], axis=1)
    xu = jnp.maximum(xu + bu_ref[...], 0.0)
    ou_ref[...] = jnp.dot(xu, Wou_ref[...], preferred_element_type=_f32) + bou_ref[...]
    xi = jnp.concatenate([mil_ref[...], mih_ref[...]], axis=1)
    xi = jnp.maximum(xi + bi_ref[...], 0.0)
    oi_ref[...] = jnp.dot(xi, Woi_ref[...], preferred_element_type=_f32) + boi_ref[...]


def _final(mu_lo, mu_hi, b_u, Wo_u, bo_u, mi_lo, mi_hi, b_i, Wo_i, bo_i):
    full = lambda s: pl.BlockSpec(s, lambda i: (0, 0))
    blk = pl.BlockSpec((BR, D), lambda i: (i, 0))
    half = pl.BlockSpec((BR, HD), lambda i: (i, 0))
    return pl.pallas_call(
        _final_body,
        grid=(NBLK,),
        in_specs=[half, half, full((1, D)), full((D, D)), full((1, D)),
                  half, half, full((1, D)), full((D, D)), full((1, D))],
        out_specs=[blk, blk],
        out_shape=[jax.ShapeDtypeStruct((N, D), _f32),
                   jax.ShapeDtypeStruct((N, D), _f32)],
    )(mu_lo, mu_hi, b_u.reshape(1, D), Wo_u, bo_u.reshape(1, D),
      mi_lo, mi_hi, b_i.reshape(1, D), Wo_i, bo_i.reshape(1, D))


def _edge_stage(w, eidx_h, als_h, ald_h, C_h, hsl_h, hsh_h,
                outl_h, outh_h, slab_h, denred_h,
                als_v, ald_v, den_v, eb_big, ed_cb, coef_v,
                gbuf, dsc, gsem, ssem, isem, tmp_r, acc_r, cvec_v,
                out_s):
    zero16 = jnp.zeros((16,), _f32)
    neg16 = jnp.full((16,), -1e30, _f32)

    pltpu.sync_copy(als_h, als_v.at[pl.ds(0, N)])
    pltpu.sync_copy(ald_h, ald_v.at[pl.ds(0, N)])
    for t in range((NP - N) // 16):
        als_v[pl.ds(N + t * 16, 16)] = neg16
        ald_v[pl.ds(N + t * 16, 16)] = neg16
    pltpu.sync_copy(C_h, cvec_v)
    cvec = cvec_v[...]

    def zden(t, _):
        den_v[pl.ds(t * 16, 16)] = zero16
        return 0
    lax.fori_loop(0, NP // 16, zden, 0)

    def p1(K, _):
        pltpu.sync_copy(eidx_h.at[w, pl.ds(K * SCH, SCH)], eb_big)
        for kk in range(SCH):
            def inner(j, _):
                s16 = eb_big[kk, 0, pl.ds(j * 16, 16)]
                d16 = eb_big[kk, 1, pl.ds(j * 16, 16)]
                a = (plsc.load_gather(als_v, [s16]) +
                     plsc.load_gather(ald_v, [d16]))
                a = jnp.where(a > 0, a, 0.2 * a)
                ex = jnp.exp(a - cvec)
                plsc.addupdate_scatter(den_v, [d16], ex)
                return 0
            lax.fori_loop(0, EC // 16, inner, 0)
        return 0
    lax.fori_loop(0, STEPS // SCH, p1, 0)

    pltpu.sync_copy(den_v, slab_h.at[w])
    plsc.subcore_barrier()
    HKW = CHK // 2
    for h in range(2):
        def zacc(t, _):
            acc_r[pl.ds(t * 16, 16)] = zero16
            return 0
        lax.fori_loop(0, HKW // 16, zacc, 0)
        for v in range(NTILES):
            pltpu.sync_copy(slab_h.at[v, pl.ds(w * CHK + h * HKW, HKW)], tmp_r)

            def addt(t, _):
                acc_r[pl.ds(t * 16, 16)] = (acc_r[pl.ds(t * 16, 16)] +
                                            tmp_r[pl.ds(t * 16, 16)])
                return 0
            lax.fori_loop(0, HKW // 16, addt, 0)
        pltpu.sync_copy(acc_r, denred_h.at[pl.ds(w * CHK + h * HKW, HKW)])
    plsc.subcore_barrier()
    pltpu.sync_copy(denred_h, den_v)

    def pco(K, _):
        pltpu.sync_copy(eidx_h.at[w, pl.ds(K * SCH, SCH)], eb_big)
        for kk in range(SCH):
            def inner(j, _):
                s16 = eb_big[kk, 0, pl.ds(j * 16, 16)]
                d16 = eb_big[kk, 1, pl.ds(j * 16, 16)]
                a = (plsc.load_gather(als_v, [s16]) +
                     plsc.load_gather(ald_v, [d16]))
                a = jnp.where(a > 0, a, 0.2 * a)
                ex = jnp.exp(a - cvec)
                dg = plsc.load_gather(den_v, [d16])
                coef_v[pl.ds((K * SCH + kk) * EC + j * 16, 16)] = (
                    ex / (dg + 1e-16))
                return 0
            lax.fori_loop(0, EC // 16, inner, 0)
        return 0
    lax.fori_loop(0, STEPS // SCH, pco, 0)

    for hs_h, out_h in ((hsl_h, outl_h), (hsh_h, outh_h)):
        def zr1(i, _):
            def zr2(l, _):
                gbuf[0][i, pl.ds(l * 16, 16)] = zero16
                return 0
            lax.fori_loop(0, HD // 16, zr2, 0)
            return 0
        lax.fori_loop(0, RB, zr1, 0)
        for b in range(CHK // RB):
            pltpu.sync_copy(gbuf[0].at[pl.ds(0, RB)],
                            out_s.at[pl.ds(w * CHK + b * RB, RB)])
        plsc.subcore_barrier()

        for i in range(2):
            pltpu.sync_copy(eidx_h.at[w, i], ed_cb.at[i, 0])
        for i in range(2):
            pltpu.async_copy(eidx_h.at[w, i + 2], ed_cb.at[i, 1], isem[i][1])
            pltpu.async_copy(hs_h.at[ed_cb.at[i, 0, 0]], gbuf[i], gsem[i])

        def grp2(hstep, _):
            for p in (0, 1):
                for i in range(2):
                    t = 4 * hstep + 2 * p + i
                    pltpu.make_async_copy(hs_h.at[ed_cb.at[i, p, 0]],
                                          gbuf[i], gsem[i]).wait()

                    def dcp(q, _):
                        dsc[i, pl.ds(q * 16, 16)] = (
                            ed_cb[i, p, 1, pl.ds(q * 16, 16)])
                        return 0
                    lax.fori_loop(0, EC // 16, dcp, 0)

                    def srow(j, _):
                        coef16 = coef_v[pl.ds(t * EC + j * 16, 16)]
                        for l in range(16):
                            e = j * 16 + l
                            cf = coef16[l]
                            for q in range(HD // 16):
                                gbuf[i][e, pl.ds(q * 16, 16)] = (
                                    gbuf[i][e, pl.ds(q * 16, 16)] * cf)
                        return 0
                    lax.fori_loop(0, EC // 16, srow, 0)

                    pltpu.async_copy(gbuf[i], out_s.at[dsc.at[i]],
                                     ssem[i], add=True)

                    @pl.when(t + 4 < STEPS)
                    def _():
                        pltpu.async_copy(eidx_h.at[w, t + 4],
                                         ed_cb.at[i, p], isem[i][p])

                    @pl.when(t + 2 < STEPS)
                    def _():
                        pltpu.make_async_copy(gbuf[i], out_s.at[dsc.at[i]],
                                              ssem[i]).wait()
                        pltpu.make_async_copy(eidx_h.at[w, t + 2],
                                              ed_cb.at[i, 1 - p],
                                              isem[i][1 - p]).wait()
                        pltpu.async_copy(hs_h.at[ed_cb.at[i, 1 - p, 0]],
                                         gbuf[i], gsem[i])
            return 0
        lax.fori_loop(0, STEPS // 4, grp2, 0)

        for i in range(2):
            pltpu.make_async_copy(gbuf[i], out_s.at[dsc.at[i]],
                                  ssem[i]).wait()
        plsc.subcore_barrier()

        @pl.when(w < NTILES - 1)
        def _():
            for b in range(CHK // RB):
                r0 = w * CHK + b * RB
                pltpu.sync_copy(out_s.at[pl.ds(r0, RB)], out_h.at[pl.ds(r0, RB)])

        @pl.when(w == NTILES - 1)
        def _():
            base15 = (NTILES - 1) * CHK
            tail = N - base15
            for b in range(3):
                r0 = base15 + b * RB
                pltpu.sync_copy(out_s.at[pl.ds(r0, RB)], out_h.at[pl.ds(r0, RB)])
            r0 = base15 + 3 * RB
            pltpu.sync_copy(out_s.at[pl.ds(r0, tail - 3 * RB)],
                            out_h.at[pl.ds(r0, tail - 3 * RB)])
        plsc.subcore_barrier()


def _edge_kernel(eidx2, als2, ald2, C2, hsl2, hsh2,
                 outl2, outh2, slab2, denred2,
                 als_v, ald_v, den_v, eb_big, ed_cb, coef_v,
                 gb0, gb1, dsc, g0, g1, s0, s1,
                 i00, i01, i10, i11,
                 tmp_r, acc_r, cvec_v, out_s):
    c = lax.axis_index("c")
    w = lax.axis_index("s")
    scr = (als_v, ald_v, den_v, eb_big, ed_cb, coef_v,
           (gb0, gb1), dsc, (g0, g1), (s0, s1),
           ((i00, i01), (i10, i11)),
           tmp_r, acc_r, cvec_v, out_s)
    _edge_stage(w, eidx2.at[c], als2.at[c], ald2.at[c], C2.at[c],
                hsl2.at[c], hsh2.at[c], outl2.at[c], outh2.at[c],
                slab2.at[c], denred2.at[c], *scr)


def _edge_call(eidx2, als2, ald2, C2, hsl2, hsh2):
    mesh = plsc.VectorSubcoreMesh(core_axis_name="c", subcore_axis_name="s")
    f = functools.partial(
        pl.kernel, mesh=mesh,
        compiler_params=pltpu.CompilerParams(needs_layout_passes=False,
                                             use_tc_tiling_on_sc=False),
        out_type=[jax.ShapeDtypeStruct((2, N, HD), _f32),
                  jax.ShapeDtypeStruct((2, N, HD), _f32),
                  jax.ShapeDtypeStruct((2, NTILES, NP), _f32),
                  jax.ShapeDtypeStruct((2, NP), _f32)],
        scratch_types=[
            pltpu.VMEM((NP,), _f32),
            pltpu.VMEM((NP,), _f32),
            pltpu.VMEM((NP,), _f32),
            pltpu.VMEM((SCH, 2, EC), jnp.int32),
            pltpu.VMEM((2, 2, 2, EC), jnp.int32),
            pltpu.VMEM((EPT,), _f32),
            pltpu.VMEM((EC, HD), _f32),
            pltpu.VMEM((EC, HD), _f32),
            pltpu.VMEM((2, EC), jnp.int32),
            pltpu.SemaphoreType.DMA,
            pltpu.SemaphoreType.DMA,
            pltpu.SemaphoreType.DMA,
            pltpu.SemaphoreType.DMA,
            pltpu.SemaphoreType.DMA,
            pltpu.SemaphoreType.DMA,
            pltpu.SemaphoreType.DMA,
            pltpu.SemaphoreType.DMA,
            pltpu.VMEM((CHK // 2,), _f32),
            pltpu.VMEM((CHK // 2,), _f32),
            pltpu.VMEM((16,), _f32),
            pltpu.VMEM_SHARED((NP, HD), _f32),
        ],
    )(_edge_kernel)
    outl2, outh2, _, _ = f(eidx2, als2, ald2, C2, hsl2, hsh2)
    return outl2[0], outh2[0], outl2[1], outh2[1]


def _pad_edges(e):
    pad = EP - E
    src = jnp.concatenate([e[0], jnp.zeros((pad,), jnp.int32)])
    dst = jnp.concatenate([e[1], jnp.full((pad,), N, jnp.int32)])
    return jnp.stack([src.reshape(NTILES, STEPS, EC),
                      dst.reshape(NTILES, STEPS, EC)], axis=2)


def _stack2(a, b):
    return jnp.stack([a, b])


def kernel(x_user, x_item, e_ui, e_iu,
           Ws_ui0, Wd_ui0, as_ui0, ad_ui0, b_ui0,
           Ws_iu0, Wd_iu0, as_iu0, ad_iu0, b_iu0,
           Ws_ui1, Wd_ui1, as_ui1, ad_ui1, b_ui1,
           Ws_iu1, Wd_iu1, as_iu1, ad_iu1, b_iu1,
           Wo_user, bo_user, Wo_item, bo_item):
    eidx2 = _stack2(_pad_edges(e_ui), _pad_edges(e_iu))
    zb = jnp.zeros((D,), _f32)
    xu_lo, xu_hi = x_user[:, :HD], x_user[:, HD:]
    xi_lo, xi_hi = x_item[:, :HD], x_item[:, HD:]

    hsl_ui, hsh_ui, als_ui, ald_ui, C_ui = _dense_pre(
        xu_lo, xu_hi, xi_lo, xi_hi, zb, zb, Ws_ui0, as_ui0, Wd_ui0, ad_ui0,
        False)
    hsl_iu, hsh_iu, als_iu, ald_iu, C_iu = _dense_pre(
        xi_lo, xi_hi, xu_lo, xu_hi, zb, zb, Ws_iu0, as_iu0, Wd_iu0, ad_iu0,
        False)
    mil0, mih0, mul0, muh0 = _edge_call(
        eidx2, _stack2(als_ui, als_iu), _stack2(ald_ui, ald_iu),
        _stack2(C_ui, C_iu), _stack2(hsl_ui, hsl_iu), _stack2(hsh_ui, hsh_iu))

    hsl1, hsh1, als_ui1, ald_ui1, C_ui1 = _dense_pre(
        mul0, muh0, mil0, mih0, b_iu0, b_ui0, Ws_ui1, as_ui1, Wd_ui1, ad_ui1,
        True)
    hsl1b, hsh1b, als_iu1, ald_iu1, C_iu1 = _dense_pre(
        mil0, mih0, mul0, muh0, b_ui0, b_iu0, Ws_iu1, as_iu1, Wd_iu1, ad_iu1,
        True)
    mil1, mih1, mul1, muh1 = _edge_call(
        eidx2, _stack2(als_ui1, als_iu1), _stack2(ald_ui1, ald_iu1),
        _stack2(C_ui1, C_iu1), _stack2(hsl1, hsl1b), _stack2(hsh1, hsh1b))

    ou, oi = _final(mul1, muh1, b_iu1, Wo_user, bo_user,
                    mil1, mih1, b_ui1, Wo_item, bo_item)
    return (ou, oi)


# speedup vs baseline: 1.0460x; 1.0460x over previous
import functools

import jax
import jax.numpy as jnp
from jax import lax
from jax.experimental import pallas as pl
from jax.experimental.pallas import tpu as pltpu
from jax.experimental.pallas import tpu_sc as plsc

N = 10000
D = 128
HD = 64
NP = 10240
E = 320000
NTILES = 16
RB = 128
CHUNKS = 160
EPT = CHUNKS * RB
EP = EPT * NTILES
CHK = NP // NTILES
NB = 4
SCH = 8
BR = 1000
NBLK = N // BR

_f32 = jnp.float32


def _dense_body(with_act, xsl_ref, xsh_ref, xdl_ref, xdh_ref, bs_ref, bd_ref,
                Ws_ref, as_ref, Wd_ref, ad_ref, hsl_ref, hsh_ref, als_ref,
                ald_ref, C_ref, mxs_ref, mxd_ref):
    i = pl.program_id(0)
    xs = jnp.concatenate([xsl_ref[...], xsh_ref[...]], axis=1)
    xd = jnp.concatenate([xdl_ref[...], xdh_ref[...]], axis=1)
    if with_act:
        xs = jnp.maximum(xs + bs_ref[...], 0.0)
        xd = jnp.maximum(xd + bd_ref[...], 0.0)
    hs = jnp.dot(xs, Ws_ref[...], preferred_element_type=_f32)
    hsl_ref[...] = hs[:, :HD]
    hsh_ref[...] = hs[:, HD:]
    hd = jnp.dot(xd, Wd_ref[...], preferred_element_type=_f32)
    als = jnp.sum(hs * as_ref[...], axis=1, keepdims=True)
    ald = jnp.sum(hd * ad_ref[...], axis=1, keepdims=True)
    als_ref[...] = als
    ald_ref[...] = ald

    @pl.when(i == 0)
    def _():
        mxs_ref[...] = jnp.full((8, 128), -3e38, _f32)
        mxd_ref[...] = jnp.full((8, 128), -3e38, _f32)

    mxs_ref[...] = jnp.maximum(mxs_ref[...], jnp.max(als))
    mxd_ref[...] = jnp.maximum(mxd_ref[...], jnp.max(ald))

    @pl.when(i == NBLK - 1)
    def _():
        C_ref[...] = jnp.zeros((8, 128), _f32) + (jnp.max(mxs_ref[...]) +
                                                  jnp.max(mxd_ref[...]))


def _dense_pre(xs_lo, xs_hi, xd_lo, xd_hi, b_src, b_dst, Ws, a_s, Wd, a_d,
               with_act):
    full = lambda s: pl.BlockSpec(s, lambda i: (0, 0))
    half = pl.BlockSpec((BR, HD), lambda i: (i, 0))
    hsl, hsh, als, ald, C = pl.pallas_call(
        functools.partial(_dense_body, with_act),
        grid=(NBLK,),
        in_specs=[half, half, half, half,
                  full((1, D)), full((1, D)), full((D, D)), full((1, D)),
                  full((D, D)), full((1, D))],
        out_specs=[half, half,
                   pl.BlockSpec((BR, 1), lambda i: (i, 0)),
                   pl.BlockSpec((BR, 1), lambda i: (i, 0)),
                   full((8, 128))],
        out_shape=[
            jax.ShapeDtypeStruct((N, HD), _f32),
            jax.ShapeDtypeStruct((N, HD), _f32),
            jax.ShapeDtypeStruct((N, 1), _f32),
            jax.ShapeDtypeStruct((N, 1), _f32),
            jax.ShapeDtypeStruct((8, 128), _f32),
        ],
        scratch_shapes=[pltpu.VMEM((8, 128), _f32), pltpu.VMEM((8, 128), _f32)],
    )(xs_lo, xs_hi, xd_lo, xd_hi, b_src.reshape(1, D), b_dst.reshape(1, D),
      Ws, a_s.reshape(1, D), Wd, a_d.reshape(1, D))
    return hsl, hsh, als.reshape(N), ald.reshape(N), C[0, :16]


def _final_body(mul_ref, muh_ref, bu_ref, Wou_ref, bou_ref, mil_ref, mih_ref,
                bi_ref, Woi_ref, boi_ref, ou_ref, oi_ref):
    xu = jnp.concatenate([mul_ref[...], muh_ref[...]], axis=1)
    xu = jnp.maximum(xu + bu_ref[...], 0.0)
    ou_ref[...] = jnp.dot(xu, Wou_ref[...], preferred_element_type=_f32) + bou_ref[...]
    xi = jnp.concatenate([mil_ref[...], mih_ref[...]], axis=1)
    xi = jnp.maximum(xi + bi_ref[...], 0.0)
    oi_ref[...] = jnp.dot(xi, Woi_ref[...], preferred_element_type=_f32) + boi_ref[...]


def _final(mu_lo, mu_hi, b_u, Wo_u, bo_u, mi_lo, mi_hi, b_i, Wo_i, bo_i):
    full = lambda s: pl.BlockSpec(s, lambda i: (0, 0))
    blk = pl.BlockSpec((BR, D), lambda i: (i, 0))
    half = pl.BlockSpec((BR, HD), lambda i: (i, 0))
    return pl.pallas_call(
        _final_body,
        grid=(NBLK,),
        in_specs=[half, half, full((1, D)), full((D, D)), full((1, D)),
                  half, half, full((1, D)), full((D, D)), full((1, D))],
        out_specs=[blk, blk],
        out_shape=[jax.ShapeDtypeStruct((N, D), _f32),
                   jax.ShapeDtypeStruct((N, D), _f32)],
    )(mu_lo, mu_hi, b_u.reshape(1, D), Wo_u, bo_u.reshape(1, D),
      mi_lo, mi_hi, b_i.reshape(1, D), Wo_i, bo_i.reshape(1, D))


def _edge_stage(w, eidx_h, als_h, ald_h, C_h, hsl_h, hsh_h,
                outl_h, outh_h, slab_h, denred_h,
                als_v, ald_v, den_v, eb_big, ed_cb, coef_v,
                gbuf, sbuf, dsc, gsem, ssem, isem, tmp_r, acc_r, cvec_v,
                out_s):
    zero16 = jnp.zeros((16,), _f32)
    neg16 = jnp.full((16,), -1e30, _f32)

    pltpu.sync_copy(als_h, als_v.at[pl.ds(0, N)])
    pltpu.sync_copy(ald_h, ald_v.at[pl.ds(0, N)])
    for t in range((NP - N) // 16):
        als_v[pl.ds(N + t * 16, 16)] = neg16
        ald_v[pl.ds(N + t * 16, 16)] = neg16
    pltpu.sync_copy(C_h, cvec_v)
    cvec = cvec_v[...]

    def zden(t, _):
        den_v[pl.ds(t * 16, 16)] = zero16
        return 0
    lax.fori_loop(0, NP // 16, zden, 0)

    def p1(K, _):
        pltpu.sync_copy(eidx_h.at[w, pl.ds(K * SCH, SCH)], eb_big)
        for kk in range(SCH):
            def inner(j, _):
                s16 = eb_big[kk, 0, pl.ds(j * 16, 16)]
                d16 = eb_big[kk, 1, pl.ds(j * 16, 16)]
                a = (plsc.load_gather(als_v, [s16]) +
                     plsc.load_gather(ald_v, [d16]))
                a = jnp.where(a > 0, a, 0.2 * a)
                ex = jnp.exp(a - cvec)
                plsc.addupdate_scatter(den_v, [d16], ex)
                return 0
            lax.fori_loop(0, RB // 16, inner, 0)
        return 0
    lax.fori_loop(0, CHUNKS // SCH, p1, 0)

    pltpu.sync_copy(den_v, slab_h.at[w])
    plsc.subcore_barrier()
    HKW = CHK // 2
    for h in range(2):
        def zacc(t, _):
            acc_r[pl.ds(t * 16, 16)] = zero16
            return 0
        lax.fori_loop(0, HKW // 16, zacc, 0)
        for v in range(NTILES):
            pltpu.sync_copy(slab_h.at[v, pl.ds(w * CHK + h * HKW, HKW)], tmp_r)

            def addt(t, _):
                acc_r[pl.ds(t * 16, 16)] = (acc_r[pl.ds(t * 16, 16)] +
                                            tmp_r[pl.ds(t * 16, 16)])
                return 0
            lax.fori_loop(0, HKW // 16, addt, 0)
        pltpu.sync_copy(acc_r, denred_h.at[pl.ds(w * CHK + h * HKW, HKW)])
    plsc.subcore_barrier()
    pltpu.sync_copy(denred_h, den_v)

    def pco(K, _):
        pltpu.sync_copy(eidx_h.at[w, pl.ds(K * SCH, SCH)], eb_big)
        for kk in range(SCH):
            def inner(j, _):
                s16 = eb_big[kk, 0, pl.ds(j * 16, 16)]
                d16 = eb_big[kk, 1, pl.ds(j * 16, 16)]
                a = (plsc.load_gather(als_v, [s16]) +
                     plsc.load_gather(ald_v, [d16]))
                a = jnp.where(a > 0, a, 0.2 * a)
                ex = jnp.exp(a - cvec)
                dg = plsc.load_gather(den_v, [d16])
                coef_v[pl.ds((K * SCH + kk) * RB + j * 16, 16)] = (
                    ex / (dg + 1e-16))
                return 0
            lax.fori_loop(0, RB // 16, inner, 0)
        return 0
    lax.fori_loop(0, CHUNKS // SCH, pco, 0)

    for hs_h, out_h in ((hsl_h, outl_h), (hsh_h, outh_h)):
        def zr1(i, _):
            def zr2(l, _):
                gbuf[0][i, pl.ds(l * 16, 16)] = zero16
                return 0
            lax.fori_loop(0, HD // 16, zr2, 0)
            return 0
        lax.fori_loop(0, RB, zr1, 0)
        for b in range(CHK // RB):
            pltpu.sync_copy(gbuf[0], out_s.at[pl.ds(w * CHK + b * RB, RB)])
        plsc.subcore_barrier()

        for i in range(2):
            pltpu.sync_copy(eidx_h.at[w, i], ed_cb.at[i, 0])
        for i in range(2):
            pltpu.async_copy(eidx_h.at[w, i + 2], ed_cb.at[i, 1], isem[i][1])
            pltpu.async_copy(hs_h.at[ed_cb.at[i, 0, 0]], gbuf[i], gsem[i])

        def grp2(hstep, _):
            for p in (0, 1):
                g = 2 * hstep + p
                for i in range(2):
                    k = g * 2 + i
                    pltpu.make_async_copy(hs_h.at[ed_cb.at[i, p, 0]],
                                          gbuf[i], gsem[i]).wait()

                    @pl.when(k >= 2)
                    def _():
                        pltpu.make_async_copy(sbuf[i], out_s.at[dsc.at[i]],
                                              ssem[i]).wait()

                    def dcp(q, _):
                        dsc[i, pl.ds(q * 16, 16)] = ed_cb[i, p, 1,
                                                          pl.ds(q * 16, 16)]
                        return 0
                    lax.fori_loop(0, RB // 16, dcp, 0)

                    def srow(j, _):
                        coef16 = coef_v[pl.ds(k * RB + j * 16, 16)]
                        for l in range(16):
                            e = j * 16 + l
                            cf = coef16[l]
                            for q in range(HD // 16):
                                sbuf[i][e, pl.ds(q * 16, 16)] = (
                                    gbuf[i][e, pl.ds(q * 16, 16)] * cf)
                        return 0
                    lax.fori_loop(0, RB // 16, srow, 0)

                    pltpu.async_copy(sbuf[i], out_s.at[dsc.at[i]],
                                     ssem[i], add=True)

                    @pl.when(k + 4 < CHUNKS)
                    def _():
                        pltpu.async_copy(eidx_h.at[w, k + 4],
                                         ed_cb.at[i, p], isem[i][p])

                    @pl.when(k + 2 < CHUNKS)
                    def _():
                        pltpu.make_async_copy(eidx_h.at[w, k + 2],
                                              ed_cb.at[i, 1 - p],
                                              isem[i][1 - p]).wait()
                        pltpu.async_copy(hs_h.at[ed_cb.at[i, 1 - p, 0]],
                                         gbuf[i], gsem[i])
            return 0
        lax.fori_loop(0, CHUNKS // 4, grp2, 0)

        for i in range(2):
            pltpu.make_async_copy(sbuf[i], out_s.at[dsc.at[i]],
                                  ssem[i]).wait()
        plsc.subcore_barrier()

        @pl.when(w < NTILES - 1)
        def _():
            for b in range(CHK // RB):
                r0 = w * CHK + b * RB
                pltpu.sync_copy(out_s.at[pl.ds(r0, RB)], out_h.at[pl.ds(r0, RB)])

        @pl.when(w == NTILES - 1)
        def _():
            base15 = (NTILES - 1) * CHK
            tail = N - base15
            for b in range(3):
                r0 = base15 + b * RB
                pltpu.sync_copy(out_s.at[pl.ds(r0, RB)], out_h.at[pl.ds(r0, RB)])
            r0 = base15 + 3 * RB
            pltpu.sync_copy(out_s.at[pl.ds(r0, tail - 3 * RB)],
                            out_h.at[pl.ds(r0, tail - 3 * RB)])
        plsc.subcore_barrier()


def _edge_kernel(eidx2, als2, ald2, C2, hsl2, hsh2,
                 outl2, outh2, slab2, denred2,
                 als_v, ald_v, den_v, eb_big, ed_cb, coef_v,
                 gb0, gb1, sb0, sb1, dsc, g0, g1, s0, s1,
                 i00, i01, i10, i11,
                 tmp_r, acc_r, cvec_v, out_s):
    c = lax.axis_index("c")
    w = lax.axis_index("s")
    scr = (als_v, ald_v, den_v, eb_big, ed_cb, coef_v,
           (gb0, gb1), (sb0, sb1), dsc, (g0, g1), (s0, s1),
           ((i00, i01), (i10, i11)),
           tmp_r, acc_r, cvec_v, out_s)
    _edge_stage(w, eidx2.at[c], als2.at[c], ald2.at[c], C2.at[c],
                hsl2.at[c], hsh2.at[c], outl2.at[c], outh2.at[c],
                slab2.at[c], denred2.at[c], *scr)


def _edge_call(eidx2, als2, ald2, C2, hsl2, hsh2):
    mesh = plsc.VectorSubcoreMesh(core_axis_name="c", subcore_axis_name="s")
    f = functools.partial(
        pl.kernel, mesh=mesh,
        compiler_params=pltpu.CompilerParams(needs_layout_passes=False,
                                             use_tc_tiling_on_sc=False),
        out_type=[jax.ShapeDtypeStruct((2, N, HD), _f32),
                  jax.ShapeDtypeStruct((2, N, HD), _f32),
                  jax.ShapeDtypeStruct((2, NTILES, NP), _f32),
                  jax.ShapeDtypeStruct((2, NP), _f32)],
        scratch_types=[
            pltpu.VMEM((NP,), _f32),
            pltpu.VMEM((NP,), _f32),
            pltpu.VMEM((NP,), _f32),
            pltpu.VMEM((SCH, 2, RB), jnp.int32),
            pltpu.VMEM((2, 2, 2, RB), jnp.int32),
            pltpu.VMEM((EPT,), _f32),
            pltpu.VMEM((RB, HD), _f32),
            pltpu.VMEM((RB, HD), _f32),
            pltpu.VMEM((RB, HD), _f32),
            pltpu.VMEM((RB, HD), _f32),
            pltpu.VMEM((2, RB), jnp.int32),
            pltpu.SemaphoreType.DMA,
            pltpu.SemaphoreType.DMA,
            pltpu.SemaphoreType.DMA,
            pltpu.SemaphoreType.DMA,
            pltpu.SemaphoreType.DMA,
            pltpu.SemaphoreType.DMA,
            pltpu.SemaphoreType.DMA,
            pltpu.SemaphoreType.DMA,
            pltpu.VMEM((CHK // 2,), _f32),
            pltpu.VMEM((CHK // 2,), _f32),
            pltpu.VMEM((16,), _f32),
            pltpu.VMEM_SHARED((NP, HD), _f32),
        ],
    )(_edge_kernel)
    outl2, outh2, _, _ = f(eidx2, als2, ald2, C2, hsl2, hsh2)
    return outl2[0], outh2[0], outl2[1], outh2[1]


def _pad_edges(e):
    pad = EP - E
    src = jnp.concatenate([e[0], jnp.zeros((pad,), jnp.int32)])
    dst = jnp.concatenate([e[1], jnp.full((pad,), N, jnp.int32)])
    return jnp.stack([src.reshape(NTILES, CHUNKS, RB),
                      dst.reshape(NTILES, CHUNKS, RB)], axis=2)


def _stack2(a, b):
    return jnp.stack([a, b])


def kernel(x_user, x_item, e_ui, e_iu,
           Ws_ui0, Wd_ui0, as_ui0, ad_ui0, b_ui0,
           Ws_iu0, Wd_iu0, as_iu0, ad_iu0, b_iu0,
           Ws_ui1, Wd_ui1, as_ui1, ad_ui1, b_ui1,
           Ws_iu1, Wd_iu1, as_iu1, ad_iu1, b_iu1,
           Wo_user, bo_user, Wo_item, bo_item):
    eidx2 = _stack2(_pad_edges(e_ui), _pad_edges(e_iu))
    zb = jnp.zeros((D,), _f32)
    xu_lo, xu_hi = x_user[:, :HD], x_user[:, HD:]
    xi_lo, xi_hi = x_item[:, :HD], x_item[:, HD:]

    hsl_ui, hsh_ui, als_ui, ald_ui, C_ui = _dense_pre(
        xu_lo, xu_hi, xi_lo, xi_hi, zb, zb, Ws_ui0, as_ui0, Wd_ui0, ad_ui0,
        False)
    hsl_iu, hsh_iu, als_iu, ald_iu, C_iu = _dense_pre(
        xi_lo, xi_hi, xu_lo, xu_hi, zb, zb, Ws_iu0, as_iu0, Wd_iu0, ad_iu0,
        False)
    mil0, mih0, mul0, muh0 = _edge_call(
        eidx2, _stack2(als_ui, als_iu), _stack2(ald_ui, ald_iu),
        _stack2(C_ui, C_iu), _stack2(hsl_ui, hsl_iu), _stack2(hsh_ui, hsh_iu))

    hsl1, hsh1, als_ui1, ald_ui1, C_ui1 = _dense_pre(
        mul0, muh0, mil0, mih0, b_iu0, b_ui0, Ws_ui1, as_ui1, Wd_ui1, ad_ui1,
        True)
    hsl1b, hsh1b, als_iu1, ald_iu1, C_iu1 = _dense_pre(
        mil0, mih0, mul0, muh0, b_ui0, b_iu0, Ws_iu1, as_iu1, Wd_iu1, ad_iu1,
        True)
    mil1, mih1, mul1, muh1 = _edge_call(
        eidx2, _stack2(als_ui1, als_iu1), _stack2(ald_ui1, ald_iu1),
        _stack2(C_ui1, C_iu1), _stack2(hsl1, hsl1b), _stack2(hsh1, hsh1b))

    ou, oi = _final(mul1, muh1, b_iu1, Wo_user, bo_user,
                    mil1, mih1, b_ui1, Wo_item, bo_item)
    return (ou, oi)
